# Initial kernel scaffold; baseline (speedup 1.0000x reference)
import jax
import jax.numpy as jnp
from jax.experimental import pallas as pl


def kernel(edge_index, node_features, edges_weight, W1, b1, g1, be1, rm1, rv1, W2, b2, g2, be2, rm2, rv2):
    raise NotImplementedError("write your pallas kernel here")


# speedup vs baseline: 15.5040x; 15.5040x over previous
import functools

import jax
import jax.numpy as jnp
from jax import lax
from jax.experimental import pallas as pl
from jax.experimental.pallas import tpu as pltpu
from jax.experimental.pallas import tpu_sc as plsc

_EPS = 1e-5
_LANES = 16
_CH = 128
_ZR = 128


def _sc_scatter(n_nodes, n_t, n_edges, n_feat):
  nb = n_edges // _CH
  info = plsc.get_sparse_core_info()
  nc, ns = info.num_cores, info.num_subcores
  passes = n_t // nc
  stripe = n_nodes // ns
  mesh = plsc.VectorSubcoreMesh(core_axis_name="c", subcore_axis_name="s")

  @functools.partial(
      pl.kernel,
      out_type=jax.ShapeDtypeStruct((n_t, n_nodes, n_feat), jnp.float32),
      mesh=mesh,
      scratch_types=[
          pltpu.VMEM((_CH,), jnp.int32),
          pltpu.VMEM((_CH,), jnp.int32),
          pltpu.VMEM((_CH,), jnp.float32),
          pltpu.VMEM((_CH, n_feat), jnp.float32),
          pltpu.VMEM((_ZR, n_feat), jnp.float32),
          pltpu.VMEM_SHARED((n_nodes, n_feat), jnp.float32),
          pltpu.SemaphoreType.DMA,
      ],
      compiler_params=pltpu.CompilerParams(
          use_tc_tiling_on_sc=False, needs_layout_passes=False),
  )
  def sc_kernel(srcs, dsts, ws, table, zeros, out,
                idx_b, dst_b, w_b, rows, zbuf, acc, sem):
    c = lax.axis_index("c")
    s = lax.axis_index("s")
    b0 = (s * nb) // ns
    b1 = ((s + 1) * nb) // ns
    r0 = s * stripe
    pltpu.sync_copy(zeros, zbuf)
    for p in range(passes):
      t = c * passes + p
      for zi in range(stripe // _ZR):
        pltpu.sync_copy(zbuf, acc.at[pl.ds(r0 + zi * _ZR, _ZR)])
      zrem = stripe % _ZR
      if zrem:
        pltpu.sync_copy(zbuf.at[pl.ds(0, zrem)],
                        acc.at[pl.ds(r0 + (stripe // _ZR) * _ZR, zrem)])
      plsc.subcore_barrier()

      def chunk_body(b, carry):
        pltpu.sync_copy(srcs.at[t, b], idx_b)
        pltpu.sync_copy(dsts.at[b], dst_b)
        pltpu.sync_copy(ws.at[t, b], w_b)
        pltpu.async_copy(table.at[idx_b], rows, sem).wait()

        def mul_body(k, mcarry):
          for u in range(4):
            e = k * 4 + u
            widx = jnp.full((_LANES,), e, dtype=jnp.int32)
            wsplat = plsc.load_gather(w_b, [widx])
            for j in range(n_feat // _LANES):
              sl = pl.ds(j * _LANES, _LANES)
              rows[e, sl] = rows[e, sl] * wsplat
          return mcarry

        lax.fori_loop(0, _CH // 4, mul_body, None)
        pltpu.sync_copy(rows, acc.at[dst_b], add=True)
        return carry

      lax.fori_loop(b0, b1, chunk_body, None)
      plsc.subcore_barrier()
      for zi in range(stripe // _CH):
        pltpu.sync_copy(acc.at[pl.ds(r0 + zi * _CH, _CH)], rows)
        pltpu.sync_copy(rows, out.at[t, pl.ds(r0 + zi * _CH, _CH)])
      crem = stripe % _CH
      if crem:
        off = r0 + (stripe // _CH) * _CH
        pltpu.sync_copy(acc.at[pl.ds(off, crem)], rows.at[pl.ds(0, crem)])
        pltpu.sync_copy(rows.at[pl.ds(0, crem)], out.at[t, pl.ds(off, crem)])

  return sc_kernel


def _matmul_body(x_ref, w_ref, o_ref):
  o_ref[...] = lax.dot_general(
      x_ref[...], w_ref[...], (((1,), (0,)), ((), ())),
      preferred_element_type=jnp.float32, precision=lax.Precision.HIGHEST)


def _tc_matmul(x, w_t, block_rows):
  n, f = x.shape
  return pl.pallas_call(
      _matmul_body,
      grid=(n // block_rows,),
      in_specs=[
          pl.BlockSpec((block_rows, f), lambda i: (i, 0)),
          pl.BlockSpec(w_t.shape, lambda i: (0, 0)),
      ],
      out_specs=pl.BlockSpec((block_rows, w_t.shape[1]), lambda i: (i, 0)),
      out_shape=jax.ShapeDtypeStruct((n, w_t.shape[1]), jnp.float32),
  )(x, w_t)


def _fused_body(a_ref, c_ref, w_ref, o_ref):
  h = jnp.maximum(a_ref[0] + c_ref[...], 0.0)
  o_ref[0] = lax.dot_general(
      h, w_ref[...], (((1,), (0,)), ((), ())),
      preferred_element_type=jnp.float32, precision=lax.Precision.HIGHEST)


def _tc_relu_matmul(a, cvec, w_t, block_rows):
  n_t, n, f = a.shape
  return pl.pallas_call(
      _fused_body,
      grid=(n_t, n // block_rows),
      in_specs=[
          pl.BlockSpec((1, block_rows, f), lambda t, i: (t, i, 0)),
          pl.BlockSpec((1, f), lambda t, i: (0, 0)),
          pl.BlockSpec((f, f), lambda t, i: (0, 0)),
      ],
      out_specs=pl.BlockSpec((1, block_rows, f), lambda t, i: (t, i, 0)),
      out_shape=jax.ShapeDtypeStruct((n_t, n, f), jnp.float32),
  )(a, cvec, w_t)


def _relu_body(a_ref, c_ref, o_ref):
  o_ref[...] = jnp.maximum(a_ref[...] + c_ref[...], 0.0)


def _tc_relu(a, cvec, block_rows):
  n_t, n, f = a.shape
  return pl.pallas_call(
      _relu_body,
      grid=(n_t, n // block_rows),
      in_specs=[
          pl.BlockSpec((1, block_rows, f), lambda t, i: (t, i, 0)),
          pl.BlockSpec((1, f), lambda t, i: (0, 0)),
      ],
      out_specs=pl.BlockSpec((1, block_rows, f), lambda t, i: (t, i, 0)),
      out_shape=jax.ShapeDtypeStruct((n_t, n, f), jnp.float32),
  )(a, cvec)


def kernel(edge_index, node_features, edges_weight,
           W1, b1, g1, be1, rm1, rv1, W2, b2, g2, be2, rm2, rv2):
  n_t, n_edges = edges_weight.shape
  n, f = node_features.shape
  src = edge_index[0]
  dst = edge_index[1]

  s1 = g1 * lax.rsqrt(rv1 + _EPS)
  w1_t = (W1 * s1[:, None]).T
  c1 = ((b1 - rm1) * s1 + be1).reshape(1, -1)
  s2 = g2 * lax.rsqrt(rv2 + _EPS)
  w2_t = (W2 * s2[:, None]).T
  c2 = ((b2 - rm2) * s2 + be2).reshape(1, -1)

  nb = n_edges // _CH
  srcs1 = jnp.broadcast_to(src, (n_t, n_edges)).reshape(n_t, nb, _CH)
  srcs2 = (src[None, :]
           + (jnp.arange(n_t, dtype=jnp.int32) * n)[:, None]).reshape(
               n_t, nb, _CH)
  dsts = dst.reshape(nb, _CH)
  ws = edges_weight.reshape(n_t, nb, _CH)
  zeros = jnp.zeros((_ZR, f), jnp.float32)

  y = _tc_matmul(node_features, w1_t, 2000)
  a1 = _sc_scatter(n, n_t, n_edges, f)(srcs1, dsts, ws, y, zeros)
  z = _tc_relu_matmul(a1, c1, w2_t, 2000)
  a2 = _sc_scatter(n, n_t, n_edges, f)(
      srcs2, dsts, ws, z.reshape(n_t * n, f), zeros)
  return _tc_relu(a2, c2, 2000)


# speedup vs baseline: 21.9349x; 1.4148x over previous
import functools

import jax
import jax.numpy as jnp
from jax import lax
from jax.experimental import pallas as pl
from jax.experimental.pallas import tpu as pltpu
from jax.experimental.pallas import tpu_sc as plsc

_EPS = 1e-5
_LANES = 16
_BLK = 128
_BPC = 2
_CH = _BLK * _BPC
_ZR = 128


def _sc_scatter(n_nodes, n_t, n_edges, n_feat):
  nb = n_edges // _BLK
  ncH = nb // _BPC
  info = plsc.get_sparse_core_info()
  nc, ns = info.num_cores, info.num_subcores
  passes = n_t // nc
  stripe = n_nodes // ns
  mesh = plsc.VectorSubcoreMesh(core_axis_name="c", subcore_axis_name="s")

  @functools.partial(
      pl.kernel,
      out_type=jax.ShapeDtypeStruct((n_t, n_nodes, n_feat), jnp.float32),
      mesh=mesh,
      scratch_types=[
          pltpu.VMEM((_BPC, _BLK), jnp.int32),
          pltpu.VMEM((_BPC, _BLK), jnp.int32),
          pltpu.VMEM((_BPC, _BLK), jnp.float32),
          pltpu.VMEM((_CH, n_feat), jnp.float32),
          pltpu.VMEM_SHARED((n_nodes, n_feat), jnp.float32),
          pltpu.SemaphoreType.DMA,
          pltpu.SemaphoreType.DMA,
          pltpu.SemaphoreType.DMA,
      ],
      compiler_params=pltpu.CompilerParams(
          use_tc_tiling_on_sc=False, needs_layout_passes=False),
  )
  def sc_kernel(srcs, dsts, ws, table, zeros, out,
                idx_b, dst_b, w_b, rows, acc, esem, gsem, ssem):
    c = lax.axis_index("c")
    s = lax.axis_index("s")
    c0 = (s * ncH) // ns
    c1 = ((s + 1) * ncH) // ns
    r0 = s * stripe
    for p in range(passes):
      t = c * passes + p
      pltpu.sync_copy(zeros, rows.at[pl.ds(0, _ZR)])
      for zi in range(stripe // _ZR):
        pltpu.sync_copy(rows.at[pl.ds(0, _ZR)],
                        acc.at[pl.ds(r0 + zi * _ZR, _ZR)])
      zrem = stripe % _ZR
      if zrem:
        pltpu.sync_copy(rows.at[pl.ds(0, zrem)],
                        acc.at[pl.ds(r0 + (stripe // _ZR) * _ZR, zrem)])
      plsc.subcore_barrier()

      def chunk_body(ci, carry):
        blk0 = ci * _BPC
        trow = t * nb + blk0
        d_idx = pltpu.async_copy(srcs.at[pl.ds(trow, _BPC)], idx_b, esem)
        d_dst = pltpu.async_copy(dsts.at[pl.ds(blk0, _BPC)], dst_b, esem)
        d_w = pltpu.async_copy(ws.at[pl.ds(trow, _BPC)], w_b, esem)
        d_idx.wait()
        gth = [
            pltpu.async_copy(table.at[idx_b.at[j]],
                             rows.at[pl.ds(j * _BLK, _BLK)], gsem)
            for j in range(_BPC)
        ]
        d_w.wait()
        for g in gth:
          g.wait()

        for j in range(_BPC):
          def mul_body(k, mcarry, j=j):
            for u in range(4):
              e = k * 4 + u
              widx = jnp.full((_LANES,), e, dtype=jnp.int32)
              wrow = jnp.full((_LANES,), j, dtype=jnp.int32)
              wsplat = plsc.load_gather(w_b, [wrow, widx])
              r = j * _BLK + e
              for f in range(n_feat // _LANES):
                sl = pl.ds(f * _LANES, _LANES)
                rows[r, sl] = rows[r, sl] * wsplat
            return mcarry

          lax.fori_loop(0, _BLK // 4, mul_body, None)

        d_dst.wait()
        sct = [
            pltpu.async_copy(rows.at[pl.ds(j * _BLK, _BLK)],
                             acc.at[dst_b.at[j]], ssem, add=True)
            for j in range(_BPC)
        ]
        for g in sct:
          g.wait()
        return carry

      lax.fori_loop(c0, c1, chunk_body, None)
      plsc.subcore_barrier()
      off = 0
      while off < stripe:
        n_r = min(_CH, stripe - off)
        pltpu.sync_copy(acc.at[pl.ds(r0 + off, n_r)], rows.at[pl.ds(0, n_r)])
        pltpu.sync_copy(rows.at[pl.ds(0, n_r)],
                        out.at[t, pl.ds(r0 + off, n_r)])
        off += n_r

  return sc_kernel


def _matmul_body(x_ref, w_ref, o_ref):
  o_ref[...] = lax.dot_general(
      x_ref[...], w_ref[...], (((1,), (0,)), ((), ())),
      preferred_element_type=jnp.float32, precision=lax.Precision.HIGHEST)


def _tc_matmul(x, w_t, block_rows):
  n, f = x.shape
  return pl.pallas_call(
      _matmul_body,
      grid=(n // block_rows,),
      in_specs=[
          pl.BlockSpec((block_rows, f), lambda i: (i, 0)),
          pl.BlockSpec(w_t.shape, lambda i: (0, 0)),
      ],
      out_specs=pl.BlockSpec((block_rows, w_t.shape[1]), lambda i: (i, 0)),
      out_shape=jax.ShapeDtypeStruct((n, w_t.shape[1]), jnp.float32),
  )(x, w_t)


def _fused_body(a_ref, c_ref, w_ref, o_ref):
  h = jnp.maximum(a_ref[0] + c_ref[...], 0.0)
  o_ref[0] = lax.dot_general(
      h, w_ref[...], (((1,), (0,)), ((), ())),
      preferred_element_type=jnp.float32, precision=lax.Precision.HIGHEST)


def _tc_relu_matmul(a, cvec, w_t, block_rows):
  n_t, n, f = a.shape
  return pl.pallas_call(
      _fused_body,
      grid=(n_t, n // block_rows),
      in_specs=[
          pl.BlockSpec((1, block_rows, f), lambda t, i: (t, i, 0)),
          pl.BlockSpec((1, f), lambda t, i: (0, 0)),
          pl.BlockSpec((f, f), lambda t, i: (0, 0)),
      ],
      out_specs=pl.BlockSpec((1, block_rows, f), lambda t, i: (t, i, 0)),
      out_shape=jax.ShapeDtypeStruct((n_t, n, f), jnp.float32),
  )(a, cvec, w_t)


def _relu_body(a_ref, c_ref, o_ref):
  o_ref[...] = jnp.maximum(a_ref[...] + c_ref[...], 0.0)


def _tc_relu(a, cvec, block_rows):
  n_t, n, f = a.shape
  return pl.pallas_call(
      _relu_body,
      grid=(n_t, n // block_rows),
      in_specs=[
          pl.BlockSpec((1, block_rows, f), lambda t, i: (t, i, 0)),
          pl.BlockSpec((1, f), lambda t, i: (0, 0)),
      ],
      out_specs=pl.BlockSpec((1, block_rows, f), lambda t, i: (t, i, 0)),
      out_shape=jax.ShapeDtypeStruct((n_t, n, f), jnp.float32),
  )(a, cvec)


def kernel(edge_index, node_features, edges_weight,
           W1, b1, g1, be1, rm1, rv1, W2, b2, g2, be2, rm2, rv2):
  n_t, n_edges = edges_weight.shape
  n, f = node_features.shape
  src = edge_index[0]
  dst = edge_index[1]

  s1 = g1 * lax.rsqrt(rv1 + _EPS)
  w1_t = (W1 * s1[:, None]).T
  c1 = ((b1 - rm1) * s1 + be1).reshape(1, -1)
  s2 = g2 * lax.rsqrt(rv2 + _EPS)
  w2_t = (W2 * s2[:, None]).T
  c2 = ((b2 - rm2) * s2 + be2).reshape(1, -1)

  nb = n_edges // _BLK
  srcs1 = jnp.broadcast_to(src, (n_t, n_edges)).reshape(n_t * nb, _BLK)
  srcs2 = (src[None, :]
           + (jnp.arange(n_t, dtype=jnp.int32) * n)[:, None]).reshape(
               n_t * nb, _BLK)
  dsts = dst.reshape(nb, _BLK)
  ws = edges_weight.reshape(n_t * nb, _BLK)
  zeros = jnp.zeros((_ZR, f), jnp.float32)

  y = _tc_matmul(node_features, w1_t, 2000)
  a1 = _sc_scatter(n, n_t, n_edges, f)(srcs1, dsts, ws, y, zeros)
  z = _tc_relu_matmul(a1, c1, w2_t, 2000)
  a2 = _sc_scatter(n, n_t, n_edges, f)(
      srcs2, dsts, ws, z.reshape(n_t * n, f), zeros)
  return _tc_relu(a2, c2, 2000)
